# baseline (speedup 1.0000x reference)
import functools

import jax
import jax.numpy as jnp
from jax import lax
from jax.experimental import pallas as pl
from jax.experimental.pallas import tpu as pltpu
from jax.experimental.pallas import tpu_sc as plsc

_TOP_K = 2
_NUM_EXPERTS = 8
_D_MODEL = 768
_N_TOKENS = 32768

_BLOCK_T = 4096
_NT_SC = 4096
_NT_TC = _N_TOKENS - _NT_SC
_SC_CORES = 2
_SC_SUBCORES = 16
_SC_WORKERS = _SC_CORES * _SC_SUBCORES
_TOK_W = _NT_SC // _SC_WORKERS
_T_CH = 16
_LANES = 16
_N_SLICE = _D_MODEL // _LANES


def _router_block(x_ref, w_ref, i1_ref, i2_ref, g1_ref, g2_ref):
    logits = jax.lax.dot_general(
        x_ref[...],
        w_ref[...],
        dimension_numbers=(((1,), (0,)), ((), ())),
        preferred_element_type=jnp.float32,
    )
    lt = logits.T
    e8 = jax.lax.broadcasted_iota(jnp.int32, lt.shape, 0)
    m1 = jnp.max(lt, axis=0, keepdims=True)
    i1 = jnp.min(jnp.where(lt == m1, e8, _NUM_EXPERTS), axis=0, keepdims=True)
    masked = jnp.where(e8 == i1, -jnp.inf, lt)
    m2 = jnp.max(masked, axis=0, keepdims=True)
    i2 = jnp.min(jnp.where(masked == m2, e8, _NUM_EXPERTS), axis=0, keepdims=True)
    ex = jnp.exp(m2 - m1)
    den = 1.0 + ex
    bt = lt.shape[1]
    i1_ref[...] = i1.reshape(1, 1, bt)
    i2_ref[...] = i2.reshape(1, 1, bt)
    g1_ref[...] = (1.0 / den).reshape(1, 1, bt)
    g2_ref[...] = (ex / den).reshape(1, 1, bt)


def _gather16(v, idx):
    return jax.lax.gather(
        v,
        idx[:, None],
        jax.lax.GatherDimensionNumbers(
            offset_dims=(), collapsed_slice_dims=(0,), start_index_map=(0,)
        ),
        (1,),
        mode=jax.lax.GatherScatterMode.PROMISE_IN_BOUNDS,
    )


def _lane_sum(v, laneseq):
    for k in (8, 4, 2, 1):
        v = v + _gather16(v, laneseq ^ k)
    return v


def _round_bf16(v):
    return v


def _sc_router(x_hbm, w_hbm, i1_hbm, i2_hbm, g1_hbm, g2_hbm,
               wv, xb, oi1, oi2, og1, og2, sem0, sem1):
    wid = lax.axis_index("s") * _SC_CORES + lax.axis_index("c")
    base = _NT_TC + wid * _TOK_W
    pltpu.sync_copy(w_hbm, wv)
    laneseq = jax.lax.broadcasted_iota(jnp.int32, (_LANES,), 0)

    sems = (sem0, sem1)
    n_ch = _TOK_W // _T_CH
    descs = [None, None]
    descs[0] = pltpu.async_copy(x_hbm.at[pl.ds(base, _T_CH), :], xb.at[0], sems[0])
    for c in range(n_ch):
        b = c % 2
        if c + 1 < n_ch:
            nxt = (c + 1) % 2
            descs[nxt] = pltpu.async_copy(
                x_hbm.at[pl.ds(base + (c + 1) * _T_CH, _T_CH), :], xb.at[nxt], sems[nxt]
            )
        descs[b].wait()

        def rb_body(rb, coll, b=b, laneseq=laneseq):
            def s_body(s, acc):
                sl = pl.ds(s * _LANES, _LANES)
                xs = [_round_bf16(xb[b, rb * 4 + t, sl]) for t in range(4)]
                ws = [wv[e, sl] for e in range(_NUM_EXPERTS)]
                return tuple(
                    tuple(acc[t][e] + xs[t] * ws[e] for e in range(_NUM_EXPERTS))
                    for t in range(4)
                )

            acc0 = tuple(
                tuple(jnp.zeros((_LANES,), jnp.float32) for _ in range(_NUM_EXPERTS))
                for _ in range(4)
            )
            acc = lax.fori_loop(0, _N_SLICE, s_body, acc0)
            for t in range(4):
                msk = laneseq == (rb * 4 + t)
                coll = tuple(
                    jnp.where(msk, _lane_sum(acc[t][e], laneseq), coll[e])
                    for e in range(_NUM_EXPERTS)
                )
            return coll

        coll0 = tuple(jnp.zeros((_LANES,), jnp.float32) for _ in range(_NUM_EXPERTS))
        ls = lax.fori_loop(0, _T_CH // 4, rb_body, coll0)

        m1 = ls[0]
        i1 = jnp.zeros((_LANES,), jnp.int32)
        m2 = jnp.full((_LANES,), -jnp.inf, jnp.float32)
        i2 = jnp.zeros((_LANES,), jnp.int32)
        for e in range(1, _NUM_EXPERTS):
            le = ls[e]
            ev = jnp.full((_LANES,), e, jnp.int32)
            gt1 = le > m1
            gt2 = le > m2
            m2 = jnp.where(gt1, m1, jnp.where(gt2, le, m2))
            i2 = jnp.where(gt1, i1, jnp.where(gt2, ev, i2))
            m1 = jnp.where(gt1, le, m1)
            i1 = jnp.where(gt1, ev, i1)
        ex = jnp.exp(m2 - m1)
        den = 1.0 + ex
        sl = pl.ds(c * _T_CH, _T_CH)
        oi1[sl] = i1
        oi2[sl] = i2
        og1[sl] = 1.0 / den
        og2[sl] = ex / den

    out_sl = pl.ds(wid * _TOK_W, _TOK_W)
    pltpu.sync_copy(oi1, i1_hbm.at[out_sl])
    pltpu.sync_copy(oi2, i2_hbm.at[out_sl])
    pltpu.sync_copy(og1, g1_hbm.at[out_sl])
    pltpu.sync_copy(og2, g2_hbm.at[out_sl])


_sc_call = functools.partial(
    pl.kernel,
    _sc_router,
    out_type=[
        jax.ShapeDtypeStruct((_NT_SC,), jnp.int32),
        jax.ShapeDtypeStruct((_NT_SC,), jnp.int32),
        jax.ShapeDtypeStruct((_NT_SC,), jnp.float32),
        jax.ShapeDtypeStruct((_NT_SC,), jnp.float32),
    ],
    mesh=plsc.VectorSubcoreMesh(
        core_axis_name="c", subcore_axis_name="s",
        num_cores=_SC_CORES, num_subcores=_SC_SUBCORES,
    ),
    scratch_types=[
        pltpu.VMEM((_NUM_EXPERTS, _D_MODEL), jnp.float32),
        pltpu.VMEM((2, _T_CH, _D_MODEL), jnp.float32),
        pltpu.VMEM((_TOK_W,), jnp.int32),
        pltpu.VMEM((_TOK_W,), jnp.int32),
        pltpu.VMEM((_TOK_W,), jnp.float32),
        pltpu.VMEM((_TOK_W,), jnp.float32),
        pltpu.SemaphoreType.DMA,
        pltpu.SemaphoreType.DMA,
    ],
)


@jax.jit
def kernel(x, W):
    n_tokens, d_model = x.shape
    wt = W.T
    wbf = W.astype(jnp.bfloat16).astype(jnp.float32)

    sc_i1, sc_i2, sc_g1, sc_g2 = _sc_call()(x, wbf)

    nb = _NT_TC // _BLOCK_T
    row_spec = pl.BlockSpec((1, 1, _BLOCK_T), lambda i: (i, 0, 0))
    row_shape_i = jax.ShapeDtypeStruct((nb, 1, _BLOCK_T), jnp.int32)
    row_shape_f = jax.ShapeDtypeStruct((nb, 1, _BLOCK_T), jnp.float32)
    i1, i2, g1, g2 = pl.pallas_call(
        _router_block,
        grid=(nb,),
        in_specs=[
            pl.BlockSpec((_BLOCK_T, d_model), lambda i: (i, 0)),
            pl.BlockSpec((d_model, _NUM_EXPERTS), lambda i: (0, 0)),
        ],
        out_specs=[row_spec, row_spec, row_spec, row_spec],
        out_shape=[row_shape_i, row_shape_i, row_shape_f, row_shape_f],
    )(x, wt)

    i1f = jnp.concatenate([i1.reshape(-1), sc_i1])
    i2f = jnp.concatenate([i2.reshape(-1), sc_i2])
    g1f = jnp.concatenate([g1.reshape(-1), sc_g1])
    g2f = jnp.concatenate([g2.reshape(-1), sc_g2])
    idx = jnp.stack([i1f, i2f], axis=1)
    gates = jnp.stack([g1f, g2f], axis=1)
    return idx, gates
